# baseline (speedup 1.0000x reference)
import functools

import jax
import jax.numpy as jnp
from jax import lax
from jax.experimental import pallas as pl
from jax.experimental.pallas import tpu as pltpu
from jax.experimental.pallas import tpu_sc as plsc

NC, NS, L = 2, 16, 16
NW = NC * NS
DIM = 64
CTX = 20
K = 6
NCH = DIM // L
NJC = 2
TBLK = 1024


def _tr_body(o_ref, ot_ref):
    ot_ref[...] = o_ref[...].T


def _transpose(o):
    d, v = o.shape
    grid = (v + TBLK - 1) // TBLK
    return pl.pallas_call(
        _tr_body,
        grid=(grid,),
        in_specs=[pl.BlockSpec((d, TBLK), lambda i: (0, i))],
        out_specs=pl.BlockSpec((TBLK, d), lambda i: (i, 0)),
        out_shape=jax.ShapeDtypeStruct((v, d), jnp.float32),
    )(o)


def _sc_forward(ctx_t, doc_ids, tn_t, d_tab, w_tab, ot_tab, *, bw):
    npass = CTX // NJC

    mesh = plsc.VectorSubcoreMesh(
        core_axis_name="c", subcore_axis_name="s",
        num_cores=NC, num_subcores=NS)

    @functools.partial(
        pl.kernel,
        out_type=jax.ShapeDtypeStruct((NW * bw * K,), jnp.float32),
        mesh=mesh,
        compiler_params=pltpu.CompilerParams(
            needs_layout_passes=False, use_tc_tiling_on_sc=False),
        scratch_types=[
            pltpu.VMEM((CTX, bw), jnp.int32),
            pltpu.VMEM((bw,), jnp.int32),
            pltpu.VMEM((K, bw), jnp.int32),
            pltpu.VMEM((2, NJC, bw, DIM), jnp.float32),
            pltpu.VMEM((bw,), jnp.int32),
            pltpu.VMEM((bw, DIM), jnp.float32),
            pltpu.VMEM_SHARED((NS * bw, DIM), jnp.float32),
            pltpu.VMEM((K, bw, DIM), jnp.float32),
            pltpu.VMEM((bw * K,), jnp.float32),
            pltpu.VMEM((L * L,), jnp.float32),
            pltpu.SemaphoreType.DMA,
            pltpu.SemaphoreType.DMA,
            pltpu.SemaphoreType.DMA,
            pltpu.SemaphoreType.DMA,
            pltpu.SemaphoreType.DMA,
        ],
    )
    def k(ctx_hbm, doc_hbm, tn_hbm, d_hbm, w_hbm, ot_hbm, out_hbm,
          ctx_g, doc_g, tn_g, w_rows, idb, x_v, xs, ot_rows, out_v, tr_v,
          sem_w, sem_d, sem_ot, sem_a0, sem_a1):
        sems_a = (sem_a0, sem_a1)
        sid = lax.axis_index("s")
        wid = sid * NC + lax.axis_index("c")
        base = wid * bw
        iota = lax.iota(jnp.int32, L)

        pltpu.sync_copy(ctx_hbm.at[:, pl.ds(base, bw)], ctx_g)
        pltpu.sync_copy(doc_hbm.at[pl.ds(base, bw)], doc_g)
        pltpu.sync_copy(tn_hbm.at[:, pl.ds(base, bw)], tn_g)

        d_cp = pltpu.async_copy(d_hbm.at[doc_g], x_v, sem_d)
        ot_cps = [
            pltpu.async_copy(ot_hbm.at[tn_g.at[kk]], ot_rows.at[kk], sem_ot)
            for kk in range(K)
        ]

        for j in range(bw // L):
            idb[pl.ds(j * L, L)] = iota + (sid * bw + j * L)

        add_q = []
        for p in range(npass):
            s = p % 2
            if p >= 2:
                for cp in add_q[:NJC]:
                    cp.wait()
                add_q = add_q[NJC:]
            w_cps = [
                pltpu.async_copy(w_hbm.at[ctx_g.at[p * NJC + t]],
                                 w_rows.at[s, t], sem_w)
                for t in range(NJC)
            ]
            if p == 0:
                d_cp.wait()
                pltpu.sync_copy(x_v, xs.at[pl.ds(sid * bw, bw)])
            for t, cp in enumerate(w_cps):
                cp.wait()
                add_q.append(
                    pltpu.async_copy(w_rows.at[s, t], xs.at[idb],
                                     sems_a[s], add=True))
        for cp in add_q:
            cp.wait()

        pltpu.sync_copy(xs.at[pl.ds(sid * bw, bw)], x_v)

        for cp in ot_cps:
            cp.wait()

        for kk in range(K):
            def dot_body(g, _, kk=kk):
                for i in range(L):
                    b = g * L + i
                    pr = x_v[b, pl.ds(0, L)] * ot_rows[kk, b, pl.ds(0, L)]
                    for c in range(1, NCH):
                        sl = pl.ds(c * L, L)
                        pr = pr + x_v[b, sl] * ot_rows[kk, b, sl]
                    plsc.store_scatter(tr_v, [iota * L + i], pr)
                s = tr_v[pl.ds(0, L)]
                for r in range(1, L):
                    s = s + tr_v[pl.ds(r * L, L)]
                out_v[pl.ds(kk * bw + g * L, L)] = s
                return 0

            lax.fori_loop(0, bw // L, dot_body, 0)

        pltpu.sync_copy(out_v, out_hbm.at[pl.ds(wid * bw * K, bw * K)])

    return k(ctx_t, doc_ids, tn_t, d_tab, w_tab, ot_tab)


@jax.jit
def _forward(context_ids, doc_ids, target_noise_ids, D, W, O):
    b = context_ids.shape[0]
    bw = b // NW
    ot = _transpose(O)
    flat = _sc_forward(context_ids.T, doc_ids, target_noise_ids.T,
                       D, W, ot, bw=bw)
    return flat.reshape(NW, K, bw).transpose(0, 2, 1).reshape(b, K)


def kernel(context_ids, doc_ids, target_noise_ids, D, W, O):
    return _forward(context_ids, doc_ids, target_noise_ids, D, W, O)


# speedup vs baseline: 1.1490x; 1.1490x over previous
import functools

import jax
import jax.numpy as jnp
from jax import lax
from jax.experimental import pallas as pl
from jax.experimental.pallas import tpu as pltpu
from jax.experimental.pallas import tpu_sc as plsc

NC, NS, L = 2, 16, 16
NW = NC * NS
DIM = 64
CTX = 20
K = 6
NCH = DIM // L
NJC = 2
TBLK = 8192


def _tr_body(o_ref, ot_ref):
    ot_ref[...] = o_ref[...].T


def _transpose(o):
    d, v = o.shape
    grid = (v + TBLK - 1) // TBLK
    return pl.pallas_call(
        _tr_body,
        grid=(grid,),
        in_specs=[pl.BlockSpec((d, TBLK), lambda i: (0, i))],
        out_specs=pl.BlockSpec((TBLK, d), lambda i: (i, 0)),
        out_shape=jax.ShapeDtypeStruct((v, d), jnp.float32),
    )(o)


def _sc_forward(ctx_t, doc_ids, tn_t, d_tab, w_tab, ot_tab, *, bw):
    npass = CTX // NJC

    mesh = plsc.VectorSubcoreMesh(
        core_axis_name="c", subcore_axis_name="s",
        num_cores=NC, num_subcores=NS)

    @functools.partial(
        pl.kernel,
        out_type=jax.ShapeDtypeStruct((NW * bw * K,), jnp.float32),
        mesh=mesh,
        compiler_params=pltpu.CompilerParams(
            needs_layout_passes=False, use_tc_tiling_on_sc=False),
        scratch_types=[
            pltpu.VMEM((CTX, bw), jnp.int32),
            pltpu.VMEM((bw,), jnp.int32),
            pltpu.VMEM((K, bw), jnp.int32),
            pltpu.VMEM((2, NJC, bw, DIM), jnp.float32),
            pltpu.VMEM((bw,), jnp.int32),
            pltpu.VMEM((bw, DIM), jnp.float32),
            pltpu.VMEM_SHARED((NS * bw, DIM), jnp.float32),
            pltpu.VMEM((K, bw, DIM), jnp.float32),
            pltpu.VMEM((bw * K,), jnp.float32),
            pltpu.VMEM((L * L,), jnp.float32),
            pltpu.SemaphoreType.DMA,
            pltpu.SemaphoreType.DMA,
            pltpu.SemaphoreType.DMA,
            pltpu.SemaphoreType.DMA,
            pltpu.SemaphoreType.DMA,
        ],
    )
    def k(ctx_hbm, doc_hbm, tn_hbm, d_hbm, w_hbm, ot_hbm, out_hbm,
          ctx_g, doc_g, tn_g, w_rows, idb, x_v, xs, ot_rows, out_v, tr_v,
          sem_w, sem_d, sem_ot, sem_a0, sem_a1):
        sems_a = (sem_a0, sem_a1)
        sid = lax.axis_index("s")
        wid = sid * NC + lax.axis_index("c")
        base = wid * bw
        iota = lax.iota(jnp.int32, L)

        pltpu.sync_copy(ctx_hbm.at[:, pl.ds(base, bw)], ctx_g)
        pltpu.sync_copy(doc_hbm.at[pl.ds(base, bw)], doc_g)
        pltpu.sync_copy(tn_hbm.at[:, pl.ds(base, bw)], tn_g)

        d_cp = pltpu.async_copy(d_hbm.at[doc_g], x_v, sem_d)
        ot_cps = [
            pltpu.async_copy(ot_hbm.at[tn_g.at[kk]], ot_rows.at[kk], sem_ot)
            for kk in range(K)
        ]

        for j in range(bw // L):
            idb[pl.ds(j * L, L)] = iota + (sid * bw + j * L)

        add_q = []
        for p in range(npass):
            s = p % 2
            if p >= 2:
                for cp in add_q[:NJC]:
                    cp.wait()
                add_q = add_q[NJC:]
            w_cps = [
                pltpu.async_copy(w_hbm.at[ctx_g.at[p * NJC + t]],
                                 w_rows.at[s, t], sem_w)
                for t in range(NJC)
            ]
            if p == 0:
                d_cp.wait()
                pltpu.sync_copy(x_v, xs.at[pl.ds(sid * bw, bw)])
            for t, cp in enumerate(w_cps):
                cp.wait()
                add_q.append(
                    pltpu.async_copy(w_rows.at[s, t], xs.at[idb],
                                     sems_a[s], add=True))
        for cp in add_q:
            cp.wait()

        pltpu.sync_copy(xs.at[pl.ds(sid * bw, bw)], x_v)

        for cp in ot_cps:
            cp.wait()

        for kk in range(K):
            def dot_body(g, _, kk=kk):
                for i in range(L):
                    b = g * L + i
                    pr = x_v[b, pl.ds(0, L)] * ot_rows[kk, b, pl.ds(0, L)]
                    for c in range(1, NCH):
                        sl = pl.ds(c * L, L)
                        pr = pr + x_v[b, sl] * ot_rows[kk, b, sl]
                    plsc.store_scatter(tr_v, [iota * L + i], pr)
                s = tr_v[pl.ds(0, L)]
                for r in range(1, L):
                    s = s + tr_v[pl.ds(r * L, L)]
                out_v[pl.ds(kk * bw + g * L, L)] = s
                return 0

            lax.fori_loop(0, bw // L, dot_body, 0)

        pltpu.sync_copy(out_v, out_hbm.at[pl.ds(wid * bw * K, bw * K)])

    return k(ctx_t, doc_ids, tn_t, d_tab, w_tab, ot_tab)


@jax.jit
def _forward(context_ids, doc_ids, target_noise_ids, D, W, O):
    b = context_ids.shape[0]
    bw = b // NW
    ot = _transpose(O)
    flat = _sc_forward(context_ids.T, doc_ids, target_noise_ids.T,
                       D, W, ot, bw=bw)
    return flat.reshape(NW, K, bw).transpose(0, 2, 1).reshape(b, K)


def kernel(context_ids, doc_ids, target_noise_ids, D, W, O):
    return _forward(context_ids, doc_ids, target_noise_ids, D, W, O)


# speedup vs baseline: 1.1560x; 1.0061x over previous
import functools

import jax
import jax.numpy as jnp
from jax import lax
from jax.experimental import pallas as pl
from jax.experimental.pallas import tpu as pltpu
from jax.experimental.pallas import tpu_sc as plsc

NC, NS, L = 2, 16, 16
NW = NC * NS
DIM = 64
CTX = 20
K = 6
NCH = DIM // L
NJC = 2
TBLK = 16384


def _tr_body(o_ref, ot_ref):
    ot_ref[...] = o_ref[...].T


def _transpose(o):
    d, v = o.shape
    grid = (v + TBLK - 1) // TBLK
    return pl.pallas_call(
        _tr_body,
        grid=(grid,),
        in_specs=[pl.BlockSpec((d, TBLK), lambda i: (0, i))],
        out_specs=pl.BlockSpec((TBLK, d), lambda i: (i, 0)),
        out_shape=jax.ShapeDtypeStruct((v, d), jnp.float32),
    )(o)


def _sc_forward(ctx_t, doc_ids, tn_t, d_tab, w_tab, ot_tab, *, bw):
    npass = CTX // NJC

    mesh = plsc.VectorSubcoreMesh(
        core_axis_name="c", subcore_axis_name="s",
        num_cores=NC, num_subcores=NS)

    @functools.partial(
        pl.kernel,
        out_type=jax.ShapeDtypeStruct((NW * bw * K,), jnp.float32),
        mesh=mesh,
        compiler_params=pltpu.CompilerParams(
            needs_layout_passes=False, use_tc_tiling_on_sc=False),
        scratch_types=[
            pltpu.VMEM((CTX, bw), jnp.int32),
            pltpu.VMEM((bw,), jnp.int32),
            pltpu.VMEM((K, bw), jnp.int32),
            pltpu.VMEM((2, NJC, bw, DIM), jnp.float32),
            pltpu.VMEM((bw,), jnp.int32),
            pltpu.VMEM((bw, DIM), jnp.float32),
            pltpu.VMEM_SHARED((NS * bw, DIM), jnp.float32),
            pltpu.VMEM((K, bw, DIM), jnp.float32),
            pltpu.VMEM((bw * K,), jnp.float32),
            pltpu.VMEM((L * L,), jnp.float32),
            pltpu.SemaphoreType.DMA,
            pltpu.SemaphoreType.DMA,
            pltpu.SemaphoreType.DMA,
            pltpu.SemaphoreType.DMA,
            pltpu.SemaphoreType.DMA,
        ],
    )
    def k(ctx_hbm, doc_hbm, tn_hbm, d_hbm, w_hbm, ot_hbm, out_hbm,
          ctx_g, doc_g, tn_g, w_rows, idb, x_v, xs, ot_rows, out_v, tr_v,
          sem_w, sem_d, sem_ot, sem_a0, sem_a1):
        sems_a = (sem_a0, sem_a1)
        sid = lax.axis_index("s")
        wid = sid * NC + lax.axis_index("c")
        base = wid * bw
        iota = lax.iota(jnp.int32, L)

        pltpu.sync_copy(ctx_hbm.at[:, pl.ds(base, bw)], ctx_g)
        pltpu.sync_copy(doc_hbm.at[pl.ds(base, bw)], doc_g)
        pltpu.sync_copy(tn_hbm.at[:, pl.ds(base, bw)], tn_g)

        d_cp = pltpu.async_copy(d_hbm.at[doc_g], x_v, sem_d)
        ot_cps = [
            pltpu.async_copy(ot_hbm.at[tn_g.at[kk]], ot_rows.at[kk], sem_ot)
            for kk in range(K)
        ]

        for j in range(bw // L):
            idb[pl.ds(j * L, L)] = iota + (sid * bw + j * L)

        add_q = []
        for p in range(npass):
            s = p % 2
            if p >= 2:
                for cp in add_q[:NJC]:
                    cp.wait()
                add_q = add_q[NJC:]
            w_cps = [
                pltpu.async_copy(w_hbm.at[ctx_g.at[p * NJC + t]],
                                 w_rows.at[s, t], sem_w)
                for t in range(NJC)
            ]
            if p == 0:
                d_cp.wait()
                pltpu.sync_copy(x_v, xs.at[pl.ds(sid * bw, bw)])
            for t, cp in enumerate(w_cps):
                cp.wait()
                add_q.append(
                    pltpu.async_copy(w_rows.at[s, t], xs.at[idb],
                                     sems_a[s], add=True))
        for cp in add_q:
            cp.wait()

        pltpu.sync_copy(xs.at[pl.ds(sid * bw, bw)], x_v)

        for cp in ot_cps:
            cp.wait()

        for kk in range(K):
            def dot_body(g, _, kk=kk):
                for i in range(L):
                    b = g * L + i
                    pr = x_v[b, pl.ds(0, L)] * ot_rows[kk, b, pl.ds(0, L)]
                    for c in range(1, NCH):
                        sl = pl.ds(c * L, L)
                        pr = pr + x_v[b, sl] * ot_rows[kk, b, sl]
                    plsc.store_scatter(tr_v, [iota * L + i], pr)
                s = tr_v[pl.ds(0, L)]
                for r in range(1, L):
                    s = s + tr_v[pl.ds(r * L, L)]
                out_v[pl.ds(kk * bw + g * L, L)] = s
                return 0

            lax.fori_loop(0, bw // L, dot_body, 0)

        pltpu.sync_copy(out_v, out_hbm.at[pl.ds(wid * bw * K, bw * K)])

    return k(ctx_t, doc_ids, tn_t, d_tab, w_tab, ot_tab)


@jax.jit
def _forward(context_ids, doc_ids, target_noise_ids, D, W, O):
    b = context_ids.shape[0]
    bw = b // NW
    ot = _transpose(O)
    flat = _sc_forward(context_ids.T, doc_ids, target_noise_ids.T,
                       D, W, ot, bw=bw)
    return flat.reshape(NW, K, bw).transpose(0, 2, 1).reshape(b, K)


def kernel(context_ids, doc_ids, target_noise_ids, D, W, O):
    return _forward(context_ids, doc_ids, target_noise_ids, D, W, O)


# speedup vs baseline: 1.2171x; 1.0529x over previous
import functools

import jax
import jax.numpy as jnp
from jax import lax
from jax.experimental import pallas as pl
from jax.experimental.pallas import tpu as pltpu
from jax.experimental.pallas import tpu_sc as plsc

NC, NS, L = 2, 16, 16
NW = NC * NS
DIM = 64
CTX = 20
K = 6
NCH = DIM // L
NJC = 2
TBLK = 16384


def _tr_body(o_ref, ot_ref):
    ot_ref[...] = o_ref[...].T


def _transpose(o):
    d, v = o.shape
    grid = (v + TBLK - 1) // TBLK
    return pl.pallas_call(
        _tr_body,
        grid=(grid,),
        in_specs=[pl.BlockSpec((d, TBLK), lambda i: (0, i))],
        out_specs=pl.BlockSpec((TBLK, d), lambda i: (i, 0)),
        out_shape=jax.ShapeDtypeStruct((v, d), jnp.float32),
    )(o)


def _sc_forward(ctx_t, doc_ids, tn_t, d_tab, w_tab, ot_tab, *, bw):
    npass = CTX // NJC

    mesh = plsc.VectorSubcoreMesh(
        core_axis_name="c", subcore_axis_name="s",
        num_cores=NC, num_subcores=NS)

    @functools.partial(
        pl.kernel,
        out_type=jax.ShapeDtypeStruct((NW * bw * K,), jnp.float32),
        mesh=mesh,
        compiler_params=pltpu.CompilerParams(
            needs_layout_passes=False, use_tc_tiling_on_sc=False),
        scratch_types=[
            pltpu.VMEM((CTX, bw), jnp.int32),
            pltpu.VMEM((bw,), jnp.int32),
            pltpu.VMEM((K, bw), jnp.int32),
            pltpu.VMEM((2, NJC, bw, DIM), jnp.float32),
            pltpu.VMEM((bw,), jnp.int32),
            pltpu.VMEM((bw, DIM), jnp.float32),
            pltpu.VMEM_SHARED((NS * bw, DIM), jnp.float32),
            pltpu.VMEM((K, bw, DIM), jnp.float32),
            pltpu.VMEM((bw * K,), jnp.float32),
            pltpu.VMEM((L * L,), jnp.float32),
            pltpu.SemaphoreType.DMA,
            pltpu.SemaphoreType.DMA,
            pltpu.SemaphoreType.DMA,
            pltpu.SemaphoreType.DMA,
            pltpu.SemaphoreType.DMA,
        ],
    )
    def k(ctx_hbm, doc_hbm, tn_hbm, d_hbm, w_hbm, ot_hbm, out_hbm,
          ctx_g, doc_g, tn_g, w_rows, idb, x_v, xs, ot_rows, out_v, tr_v,
          sem_w, sem_d, sem_ot, sem_a0, sem_a1):
        sems_a = (sem_a0, sem_a1)
        sid = lax.axis_index("s")
        wid = sid * NC + lax.axis_index("c")
        base = wid * bw
        iota = lax.iota(jnp.int32, L)

        pltpu.sync_copy(ctx_hbm.at[:, pl.ds(base, bw)], ctx_g)
        pltpu.sync_copy(doc_hbm.at[pl.ds(base, bw)], doc_g)
        pltpu.sync_copy(tn_hbm.at[:, pl.ds(base, bw)], tn_g)

        d_cp = pltpu.async_copy(d_hbm.at[doc_g], x_v, sem_d)
        ot_cps = [
            pltpu.async_copy(ot_hbm.at[tn_g.at[kk]], ot_rows.at[kk], sem_ot)
            for kk in range(K)
        ]

        for j in range(bw // L):
            idb[pl.ds(j * L, L)] = iota + (sid * bw + j * L)

        add_q = []
        for p in range(npass):
            s = p % 2
            if p >= 2:
                for cp in add_q[:NJC]:
                    cp.wait()
                add_q = add_q[NJC:]
            w_cps = [
                pltpu.async_copy(w_hbm.at[ctx_g.at[p * NJC + t]],
                                 w_rows.at[s, t], sem_w)
                for t in range(NJC)
            ]
            if p == 0:
                d_cp.wait()
                pltpu.sync_copy(x_v, xs.at[pl.ds(sid * bw, bw)])
            for t, cp in enumerate(w_cps):
                cp.wait()
                add_q.append(
                    pltpu.async_copy(w_rows.at[s, t], xs.at[idb],
                                     sems_a[s], add=True))
        for cp in add_q:
            cp.wait()

        pltpu.sync_copy(xs.at[pl.ds(sid * bw, bw)], x_v)

        for cp in ot_cps:
            cp.wait()

        for kk in range(K):
            def dot_body(g, _, kk=kk):
                for i in range(L):
                    b = g * L + i
                    pr = x_v[b, pl.ds(0, L)] * ot_rows[kk, b, pl.ds(0, L)]
                    for c in range(1, NCH):
                        sl = pl.ds(c * L, L)
                        pr = pr + x_v[b, sl] * ot_rows[kk, b, sl]
                    plsc.store_scatter(tr_v, [iota * L + i], pr)
                s = tr_v[pl.ds(0, L)]
                for r in range(1, L):
                    s = s + tr_v[pl.ds(r * L, L)]
                out_v[pl.ds(kk * bw + g * L, L)] = s
                return 0

            lax.fori_loop(0, bw // L, dot_body, 0)

        pltpu.sync_copy(out_v, out_hbm.at[pl.ds(wid * bw * K, bw * K)])

    return k(ctx_t, doc_ids, tn_t, d_tab, w_tab, ot_tab)


@jax.jit
def _forward(context_ids, doc_ids, target_noise_ids, D, W, O):
    b = context_ids.shape[0]
    bw = b // NW
    ot = O.T.reshape(-1, DIM)
    flat = _sc_forward(context_ids.T, doc_ids, target_noise_ids.T,
                       D, W, ot, bw=bw)
    return flat.reshape(NW, K, bw).transpose(0, 2, 1).reshape(b, K)


def kernel(context_ids, doc_ids, target_noise_ids, D, W, O):
    return _forward(context_ids, doc_ids, target_noise_ids, D, W, O)


# speedup vs baseline: 1.2247x; 1.0063x over previous
import functools

import jax
import jax.numpy as jnp
from jax import lax
from jax.experimental import pallas as pl
from jax.experimental.pallas import tpu as pltpu
from jax.experimental.pallas import tpu_sc as plsc

NC, NS, L = 2, 16, 16
NW = NC * NS
DIM = 64
CTX = 20
K = 6
NCH = DIM // L
NJC = 2
TBLK = 16384


def _tr_body(o_ref, ot_ref):
    ot_ref[...] = o_ref[...].T


def _transpose(o):
    d, v = o.shape
    grid = (v + TBLK - 1) // TBLK
    return pl.pallas_call(
        _tr_body,
        grid=(grid,),
        in_specs=[pl.BlockSpec((d, TBLK), lambda i: (0, i))],
        out_specs=pl.BlockSpec((TBLK, d), lambda i: (i, 0)),
        out_shape=jax.ShapeDtypeStruct((v, d), jnp.float32),
    )(o)


def _sc_forward(ctx_t, doc_ids, tn_t, d_tab, w_tab, ot_tab, *, bw):
    npass = CTX // NJC

    mesh = plsc.VectorSubcoreMesh(
        core_axis_name="c", subcore_axis_name="s",
        num_cores=NC, num_subcores=NS)

    @functools.partial(
        pl.kernel,
        out_type=jax.ShapeDtypeStruct((NW * bw * K,), jnp.float32),
        mesh=mesh,
        compiler_params=pltpu.CompilerParams(
            needs_layout_passes=False, use_tc_tiling_on_sc=False),
        scratch_types=[
            pltpu.VMEM((CTX, bw), jnp.int32),
            pltpu.VMEM((bw,), jnp.int32),
            pltpu.VMEM((K, bw), jnp.int32),
            pltpu.VMEM((2, NJC, bw, DIM), jnp.float32),
            pltpu.VMEM((bw,), jnp.int32),
            pltpu.VMEM((bw, DIM), jnp.float32),
            pltpu.VMEM_SHARED((NS * bw, DIM), jnp.float32),
            pltpu.VMEM((K, bw, DIM), jnp.float32),
            pltpu.VMEM((bw * K,), jnp.float32),
            pltpu.VMEM((L * L,), jnp.float32),
            pltpu.SemaphoreType.DMA,
            pltpu.SemaphoreType.DMA,
            pltpu.SemaphoreType.DMA,
            pltpu.SemaphoreType.DMA,
            pltpu.SemaphoreType.DMA,
        ],
    )
    def k(ctx_hbm, doc_hbm, tn_hbm, d_hbm, w_hbm, ot_hbm, out_hbm,
          ctx_g, doc_g, tn_g, w_rows, idb, x_v, xs, ot_rows, out_v, tr_v,
          sem_w, sem_d, sem_ot, sem_a0, sem_a1):
        sems_a = (sem_a0, sem_a1)
        sid = lax.axis_index("s")
        wid = sid * NC + lax.axis_index("c")
        base = wid * bw
        iota = lax.iota(jnp.int32, L)

        st_cps = [
            pltpu.async_copy(ctx_hbm.at[:, pl.ds(base, bw)], ctx_g, sem_a0),
            pltpu.async_copy(doc_hbm.at[pl.ds(base, bw)], doc_g, sem_a0),
            pltpu.async_copy(tn_hbm.at[:, pl.ds(base, bw)], tn_g, sem_a0),
        ]
        for cp in st_cps:
            cp.wait()

        d_cp = pltpu.async_copy(d_hbm.at[doc_g], x_v, sem_d)
        ot_cps = [
            pltpu.async_copy(ot_hbm.at[tn_g.at[kk]], ot_rows.at[kk], sem_ot)
            for kk in range(K)
        ]

        for j in range(bw // L):
            idb[pl.ds(j * L, L)] = iota + (sid * bw + j * L)

        add_q = []
        for p in range(npass):
            s = p % 2
            if p >= 2:
                for cp in add_q[:NJC]:
                    cp.wait()
                add_q = add_q[NJC:]
            w_cps = [
                pltpu.async_copy(w_hbm.at[ctx_g.at[p * NJC + t]],
                                 w_rows.at[s, t], sem_w)
                for t in range(NJC)
            ]
            if p == 0:
                d_cp.wait()
                pltpu.sync_copy(x_v, xs.at[pl.ds(sid * bw, bw)])
            for t, cp in enumerate(w_cps):
                cp.wait()
                add_q.append(
                    pltpu.async_copy(w_rows.at[s, t], xs.at[idb],
                                     sems_a[s], add=True))
        for cp in add_q:
            cp.wait()

        pltpu.sync_copy(xs.at[pl.ds(sid * bw, bw)], x_v)

        for cp in ot_cps:
            cp.wait()

        for kk in range(K):
            def dot_body(g, _, kk=kk):
                for i in range(L):
                    b = g * L + i
                    pr = x_v[b, pl.ds(0, L)] * ot_rows[kk, b, pl.ds(0, L)]
                    for c in range(1, NCH):
                        sl = pl.ds(c * L, L)
                        pr = pr + x_v[b, sl] * ot_rows[kk, b, sl]
                    plsc.store_scatter(tr_v, [iota * L + i], pr)
                s = tr_v[pl.ds(0, L)]
                for r in range(1, L):
                    s = s + tr_v[pl.ds(r * L, L)]
                out_v[pl.ds(kk * bw + g * L, L)] = s
                return 0

            lax.fori_loop(0, bw // L, dot_body, 0)

        pltpu.sync_copy(out_v, out_hbm.at[pl.ds(wid * bw * K, bw * K)])

    return k(ctx_t, doc_ids, tn_t, d_tab, w_tab, ot_tab)


@jax.jit
def _forward(context_ids, doc_ids, target_noise_ids, D, W, O):
    b = context_ids.shape[0]
    bw = b // NW
    ot = O.T.reshape(-1, DIM)
    flat = _sc_forward(context_ids.T, doc_ids, target_noise_ids.T,
                       D, W, ot, bw=bw)
    return flat.reshape(NW, K, bw).transpose(0, 2, 1).reshape(b, K)


def kernel(context_ids, doc_ids, target_noise_ids, D, W, O):
    return _forward(context_ids, doc_ids, target_noise_ids, D, W, O)
